# baseline (speedup 1.0000x reference)
import functools
import math

import jax
import jax.numpy as jnp
from jax import lax
from jax.experimental import pallas as pl
from jax.experimental.pallas import tpu as pltpu
from jax.experimental.pallas import tpu_sc as plsc

N = 10000
F = 64
K = 32
DK = 16
F2 = 2 * F
EMB_SPLIT = 64
ALPHA = 0.1
INV_SCALE = 1.0 / math.sqrt(DK)

NW = 32
NPW = 320
NP = NW * NPW
CHUNK = 128
NCHUNK = NPW * K // CHUNK

G = 128
S = 16

_HI = jax.lax.Precision.DEFAULT
_f32 = jnp.float32


@functools.lru_cache(maxsize=None)
def _make_sc_gather(D, nchunk):
  mesh = plsc.VectorSubcoreMesh(core_axis_name="c", subcore_axis_name="s")

  @functools.partial(
      pl.kernel,
      out_type=jax.ShapeDtypeStruct((NW * nchunk * CHUNK, D), _f32),
      mesh=mesh,
      scratch_types=[
          pltpu.VMEM((nchunk, CHUNK), jnp.int32),
          pltpu.VMEM((4, CHUNK, D), _f32),
          pltpu.SemaphoreType.DMA,
          pltpu.SemaphoreType.DMA,
          pltpu.SemaphoreType.DMA,
          pltpu.SemaphoreType.DMA,
          pltpu.SemaphoreType.DMA,
          pltpu.SemaphoreType.DMA,
          pltpu.SemaphoreType.DMA,
          pltpu.SemaphoreType.DMA,
      ],
      compiler_params=pltpu.CompilerParams(use_tc_tiling_on_sc=False),
  )
  def sc_gather(idx_hbm, tab_hbm, out_hbm, idx_v, rows, sg0, sg1, sg2, sg3,
                ss0, ss1, ss2, ss3):
    wid = lax.axis_index("s") * 2 + lax.axis_index("c")
    pltpu.sync_copy(idx_hbm.at[wid], idx_v)
    base = wid * (nchunk * CHUNK)
    sgs = [sg0, sg1, sg2, sg3]
    sss = [ss0, ss1, ss2, ss3]

    def out_at(c):
      return out_hbm.at[pl.ds(base + c * CHUNK, CHUNK)]

    def body(t, carry):
      @pl.when(t > 0)
      def _():
        for q in range(4):
          pltpu.make_async_copy(rows.at[q], out_at(4 * t + q - 4),
                                sss[q]).wait()

      for q in range(4):
        pltpu.async_copy(tab_hbm.at[idx_v.at[4 * t + q]], rows.at[q], sgs[q])
      for q in range(4):
        c = 4 * t + q
        pltpu.make_async_copy(tab_hbm.at[idx_v.at[c]], rows.at[q],
                              sgs[q]).wait()
        pltpu.async_copy(rows.at[q], out_at(c), sss[q])
      return carry

    lax.fori_loop(0, nchunk // 4, body, 0)
    for q in range(4):
      pltpu.make_async_copy(rows.at[q], out_at(nchunk - 4 + q), sss[q]).wait()

  return sc_gather


def _sc_gather_x(kadj_r, tab):
  return _make_sc_gather(F, kadj_r.shape[1])(kadj_r, tab)


NB = 4


def _lane_bcast(v, lane):
  return lax.gather(
      v, jnp.full((16, 1), lane, jnp.int32),
      lax.GatherDimensionNumbers(offset_dims=(), collapsed_slice_dims=(0,),
                                 start_index_map=(0,)),
      (1,), mode=lax.GatherScatterMode.PROMISE_IN_BOUNDS)


def _bsum(v):
  return _lane_bcast(plsc.cumsum(v), 15)


@functools.lru_cache(maxsize=None)
def _make_sc_cell():
  mesh = plsc.VectorSubcoreMesh(core_axis_name="c", subcore_axis_name="s")

  @functools.partial(
      pl.kernel,
      out_type=jax.ShapeDtypeStruct((NP, F2), _f32),
      mesh=mesh,
      scratch_types=[
          pltpu.VMEM((NCHUNK, CHUNK), jnp.int32),
          pltpu.VMEM((4, CHUNK, F2), _f32),
          pltpu.VMEM((NP,), _f32),
          pltpu.VMEM((NP,), _f32),
          pltpu.VMEM((4, NB, F2), _f32),
          pltpu.VMEM((4, NB, F2), _f32),
          pltpu.VMEM((F2,), _f32),
          pltpu.VMEM((F2,), _f32),
      ] + [pltpu.SemaphoreType.DMA] * 12,
      compiler_params=pltpu.CompilerParams(use_tc_tiling_on_sc=False,
                                           needs_layout_passes=False),
  )
  def sc_cell(idx_hbm, hall_hbm, auxt_hbm, gam_hbm, bet_hbm, out_hbm,
              idx_v, rows, w1t, w2t, own_v, outb, gam_v, bet_v, *sems):
    sgs = sems[0:4]
    sos = sems[4:8]
    sus = sems[8:12]
    wid = lax.axis_index("s") * 2 + lax.axis_index("c")
    base = wid * NPW
    pltpu.sync_copy(idx_hbm.at[wid], idx_v)
    pltpu.sync_copy(auxt_hbm.at[0], w1t)
    pltpu.sync_copy(auxt_hbm.at[1], w2t)
    pltpu.sync_copy(gam_hbm, gam_v)
    pltpu.sync_copy(bet_hbm, bet_v)

    def own_at(c):
      return hall_hbm.at[pl.ds(base + c * NB, NB)]

    def out_at(c):
      return out_hbm.at[pl.ds(base + c * NB, NB)]

    def process(c, q):
      pltpu.make_async_copy(own_at(c), own_v.at[q], sos[q]).wait()
      for b in range(NB):
        gidx = base + c * NB + b
        iv0 = idx_v[c, pl.ds(b * K, 16)]
        iv1 = idx_v[c, pl.ds(b * K + 16, 16)]
        w2s = plsc.load_gather(w2t, [jnp.full((16,), gidx, jnp.int32)])
        e0 = plsc.load_gather(w1t, [iv0]) + w2s
        e1 = plsc.load_gather(w1t, [iv1]) + w2s
        e0 = jnp.where(e0 > 0, e0, ALPHA * e0)
        e1 = jnp.where(e1 > 0, e1, ALPHA * e1)
        x0 = jnp.exp(e0)
        x1 = jnp.exp(e1)
        tot = _bsum(x0 + x1)
        a0 = x0 / tot
        a1 = x1 / tot
        acc = [jnp.zeros((16,), _f32) for _ in range(F2 // 16)]
        for k in range(K):
          wk = _lane_bcast(a0 if k < 16 else a1, k % 16)
          r = b * K + k
          for j in range(F2 // 16):
            acc[j] = acc[j] + wk * rows[q, r, pl.ds(j * 16, 16)]
        sv = jnp.zeros((16,), _f32)
        qv = jnp.zeros((16,), _f32)
        for j in range(F2 // 16):
          o = acc[j] + own_v[q, b, pl.ds(j * 16, 16)]
          o = jnp.where(o > 0, o, ALPHA * o)
          acc[j] = o
          sv = sv + o
          qv = qv + o * o
        mu = _bsum(sv) * (1.0 / F2)
        var = _bsum(qv) * (1.0 / F2) - mu * mu
        t = var + 1e-5
        ti = plsc.bitcast(t, jnp.int32)
        yi = jnp.int32(0x5F3759DF) - lax.shift_right_logical(ti, 1)
        y = plsc.bitcast(yi, _f32)
        for _ in range(3):
          y = y * (1.5 - 0.5 * t * y * y)
        for j in range(F2 // 16):
          g = gam_v[pl.ds(j * 16, 16)]
          bb = bet_v[pl.ds(j * 16, 16)]
          outb[q, b, pl.ds(j * 16, 16)] = (acc[j] - mu) * y * g + bb
      pltpu.async_copy(outb.at[q], out_at(c), sus[q])

    def gat(c, q):
      pltpu.async_copy(hall_hbm.at[idx_v.at[c]], rows.at[q], sgs[q])
      pltpu.async_copy(own_at(c), own_v.at[q], sos[q])

    gat(0, 0)
    gat(1, 1)
    gat(2, 2)

    def body(t, carry):
      for q in range(4):
        c = 4 * t + q

        @pl.when(c >= 4)
        def _(c=c, q=q):
          pltpu.make_async_copy(outb.at[q], out_at(c - 4), sus[q]).wait()

        pltpu.make_async_copy(hall_hbm.at[idx_v.at[c]], rows.at[q],
                              sgs[q]).wait()
        process(c, q)

        @pl.when(c + 3 < NCHUNK)
        def _(c=c, q=q):
          gat(c + 3, (q + 3) % 4)
      return carry

    lax.fori_loop(0, NCHUNK // 4, body, 0)
    for q in range(4):
      pltpu.make_async_copy(outb.at[q], out_at(NCHUNK - 4 + q), sus[q]).wait()

  return sc_cell


def _sc_cell(kadj_r, h_all, auxt, gam, bet):
  return _make_sc_cell()(kadj_r, h_all, auxt, gam, bet)


def _tcb_body(x_ref, hk_ref, r2_ref, wcol_ref, bcol_ref, wall_ref, ball_ref,
              c18_ref, hall_ref, auxt_ref, aux_scr):
  r2 = r2_ref[...]
  wcol = wcol_ref[...]
  bcol = bcol_ref[...]
  wall = wall_ref[...]
  ball = ball_ref[...]
  rown = lax.broadcasted_iota(jnp.int32, (S, S * F), 0)
  coln = lax.broadcasted_iota(jnp.int32, (S, S * F), 1) // F
  maskx = rown == coln
  onesbd = jnp.where(maskx, 1.0, 0.0).astype(_f32)

  def sub(i, carry):
    xs = x_ref[pl.ds(i * S, S), :]
    x_rep = jnp.dot(r2, xs, precision=_HI,
                    preferred_element_type=_f32)
    wht = jax.nn.relu(wcol * x_rep + bcol)
    big = jnp.dot(wall.astype(jnp.bfloat16), wht.astype(jnp.bfloat16),
                  precision=_HI,
                  preferred_element_type=_f32) + ball
    q3 = big[0:S * DK].reshape(S, DK, F)
    k3 = big[S * DK:2 * S * DK].reshape(S, DK, F)
    p3 = big[2 * S * DK:].reshape(S, K, F)
    hk3 = hk_ref[pl.ds(i * S * K, S * K), :].reshape(S, K, F)

    lre = lax.dot_general(k3, q3, (((1,), (1,)), ((0,), (0,))),
                          precision=_HI, preferred_element_type=_f32)
    lcc = lax.dot_general(hk3, p3, (((1,), (1,)), ((0,), (0,))),
                          precision=_HI, preferred_element_type=_f32)
    ere = jnp.exp(lre.reshape(S * F, F))
    ecc = jnp.exp(lcc.reshape(S * F, F))
    xbd = jnp.where(maskx, jnp.tile(xs, (1, S)), 0.0)
    wsel = jnp.concatenate([xbd, onesbd], axis=0)
    outre = jnp.dot(wsel, ere, precision=_HI,
                    preferred_element_type=_f32)
    outcc = jnp.dot(wsel, ecc, precision=_HI,
                    preferred_element_type=_f32)
    hre = outre[0:S] / outre[S:2 * S] + xs
    hcc = outcc[0:S] / outcc[S:2 * S] + xs
    hall_s = jnp.concatenate([hre, hcc], axis=1)
    hall_ref[pl.ds(i * S, S), :] = hall_s
    aux = jnp.dot(hall_s, c18_ref[...], precision=_HI,
                  preferred_element_type=_f32)
    aux_scr[pl.ds(i * S, S), :] = aux
    return carry

  lax.fori_loop(0, G // S, sub, 0)
  auxt_ref[...] = lax.transpose(aux_scr[...], (1, 0))


def _tc_dense(xp, hk, r2, wcol, bcol, wall, ball, c18):
  nn = xp.shape[0]
  wspec = lambda shape: pl.BlockSpec(shape, lambda i: (0, 0))
  return pl.pallas_call(
      _tcb_body,
      grid=(nn // G,),
      in_specs=[
          pl.BlockSpec((G, F), lambda i: (i, 0)),
          pl.BlockSpec((G * K, F), lambda i: (i, 0)),
          wspec((S * DK, S)), wspec((S * DK, 1)), wspec((S * DK, 1)),
          wspec((S * (2 * DK + K), S * DK)), wspec((S * (2 * DK + K), 1)),
          wspec((F2, 8)),
      ],
      out_specs=[
          pl.BlockSpec((G, F2), lambda i: (i, 0)),
          pl.BlockSpec((8, G), lambda i: (0, i)),
      ],
      out_shape=[
          jax.ShapeDtypeStruct((nn, F2), _f32),
          jax.ShapeDtypeStruct((8, nn), _f32),
      ],
      scratch_shapes=[pltpu.VMEM((G, 8), _f32)],
  )(xp, hk, r2, wcol, bcol, wall, ball, c18)


def kernel(x, kadj, Wh_w, Wh_b, Wq, bq, Wk, bk, a_gene_cc, W_cell_cc,
           a_cell_cc, ln_gamma, ln_beta):
  x = x.astype(_f32)
  kadj = kadj.astype(jnp.int32)

  xp = jnp.zeros((NP, F), _f32).at[:N].set(x)
  kadjp = jnp.zeros((NP, K), jnp.int32).at[:N].set(kadj)
  kadj_r = kadjp.reshape(NW, NCHUNK, CHUNK)

  eye_s = jnp.eye(S, dtype=_f32)
  r2 = jnp.kron(eye_s, jnp.ones((DK, 1), _f32))
  wcol = jnp.tile(Wh_w[0], S)[:, None].astype(_f32)
  bcol = jnp.tile(Wh_b, S)[:, None].astype(_f32)
  wqtk = jnp.kron(eye_s, Wq.T.astype(_f32)) * INV_SCALE
  bqcol = (jnp.tile(bq, S)[:, None] * INV_SCALE).astype(_f32)
  wktk = jnp.kron(eye_s, Wk.T.astype(_f32))
  bkcol = jnp.tile(bk, S)[:, None].astype(_f32)
  agtk = jnp.kron(eye_s, a_gene_cc.T.astype(_f32))
  wall = jnp.concatenate([wqtk, wktk, agtk @ wqtk], axis=0)
  ball = jnp.concatenate([bqcol, bkcol, agtk @ bqcol], axis=0)

  c1 = (W_cell_cc @ a_cell_cc[:EMB_SPLIT]).astype(_f32)
  c2 = (W_cell_cc @ a_cell_cc[EMB_SPLIT:]).astype(_f32)
  c18 = jnp.concatenate([c1, c2, jnp.zeros((F2, 6), _f32)], axis=1)

  half = NP // 2
  nchunk_h = half * K // (NW * CHUNK)
  kadj_lo = kadjp[:half].reshape(NW, nchunk_h, CHUNK)
  kadj_hi = kadjp[half:].reshape(NW, nchunk_h, CHUNK)
  hk_lo = _sc_gather_x(kadj_lo, x)
  hk_hi = _sc_gather_x(kadj_hi, x)
  h_lo, aux_lo = _tc_dense(xp[:half], hk_lo, r2, wcol, bcol, wall, ball, c18)
  h_hi, aux_hi = _tc_dense(xp[half:], hk_hi, r2, wcol, bcol, wall, ball, c18)
  h_all = jnp.concatenate([h_lo, h_hi], axis=0)
  auxt = jnp.concatenate([aux_lo, aux_hi], axis=1)
  out = _sc_cell(kadj_r, h_all, auxt, ln_gamma.astype(_f32),
                 ln_beta.astype(_f32))
  return out[:N]


# speedup vs baseline: 1.0727x; 1.0727x over previous
import functools
import math

import jax
import jax.numpy as jnp
from jax import lax
from jax.experimental import pallas as pl
from jax.experimental.pallas import tpu as pltpu
from jax.experimental.pallas import tpu_sc as plsc

N = 10000
F = 64
K = 32
DK = 16
F2 = 2 * F
EMB_SPLIT = 64
ALPHA = 0.1
INV_SCALE = 1.0 / math.sqrt(DK)

NW = 32
NPW = 320
NP = NW * NPW
CHUNK = 128
NCHUNK = NPW * K // CHUNK

G = 256
S = 16
S4 = 4

_HI = jax.lax.Precision.DEFAULT
_f32 = jnp.float32


@functools.lru_cache(maxsize=None)
def _make_sc_gather(D, nchunk):
  mesh = plsc.VectorSubcoreMesh(core_axis_name="c", subcore_axis_name="s")

  @functools.partial(
      pl.kernel,
      out_type=jax.ShapeDtypeStruct((NW * nchunk * CHUNK, D), _f32),
      mesh=mesh,
      scratch_types=[
          pltpu.VMEM((nchunk, CHUNK), jnp.int32),
          pltpu.VMEM((4, CHUNK, D), _f32),
          pltpu.SemaphoreType.DMA,
          pltpu.SemaphoreType.DMA,
          pltpu.SemaphoreType.DMA,
          pltpu.SemaphoreType.DMA,
          pltpu.SemaphoreType.DMA,
          pltpu.SemaphoreType.DMA,
          pltpu.SemaphoreType.DMA,
          pltpu.SemaphoreType.DMA,
      ],
      compiler_params=pltpu.CompilerParams(use_tc_tiling_on_sc=False),
  )
  def sc_gather(idx_hbm, tab_hbm, out_hbm, idx_v, rows, sg0, sg1, sg2, sg3,
                ss0, ss1, ss2, ss3):
    wid = lax.axis_index("s") * 2 + lax.axis_index("c")
    pltpu.sync_copy(idx_hbm.at[wid], idx_v)
    base = wid * (nchunk * CHUNK)
    sgs = [sg0, sg1, sg2, sg3]
    sss = [ss0, ss1, ss2, ss3]

    def out_at(c):
      return out_hbm.at[pl.ds(base + c * CHUNK, CHUNK)]

    def body(t, carry):
      @pl.when(t > 0)
      def _():
        for q in range(4):
          pltpu.make_async_copy(rows.at[q], out_at(4 * t + q - 4),
                                sss[q]).wait()

      for q in range(4):
        pltpu.async_copy(tab_hbm.at[idx_v.at[4 * t + q]], rows.at[q], sgs[q])
      for q in range(4):
        c = 4 * t + q
        pltpu.make_async_copy(tab_hbm.at[idx_v.at[c]], rows.at[q],
                              sgs[q]).wait()
        pltpu.async_copy(rows.at[q], out_at(c), sss[q])
      return carry

    lax.fori_loop(0, nchunk // 4, body, 0)
    for q in range(4):
      pltpu.make_async_copy(rows.at[q], out_at(nchunk - 4 + q), sss[q]).wait()

  return sc_gather


def _sc_gather_x(kadj_r, tab):
  return _make_sc_gather(F, kadj_r.shape[1])(kadj_r, tab)


NB = 4


def _lane_bcast(v, lane):
  return lax.gather(
      v, jnp.full((16, 1), lane, jnp.int32),
      lax.GatherDimensionNumbers(offset_dims=(), collapsed_slice_dims=(0,),
                                 start_index_map=(0,)),
      (1,), mode=lax.GatherScatterMode.PROMISE_IN_BOUNDS)


def _bsum(v):
  return _lane_bcast(plsc.cumsum(v), 15)


@functools.lru_cache(maxsize=None)
def _make_sc_cell():
  mesh = plsc.VectorSubcoreMesh(core_axis_name="c", subcore_axis_name="s")

  @functools.partial(
      pl.kernel,
      out_type=jax.ShapeDtypeStruct((NP, F2), _f32),
      mesh=mesh,
      scratch_types=[
          pltpu.VMEM((NCHUNK, CHUNK), jnp.int32),
          pltpu.VMEM((4, CHUNK, F2), _f32),
          pltpu.VMEM((NP,), _f32),
          pltpu.VMEM((NP,), _f32),
          pltpu.VMEM((4, NB, F2), _f32),
          pltpu.VMEM((4, NB, F2), _f32),
          pltpu.VMEM((F2,), _f32),
          pltpu.VMEM((F2,), _f32),
      ] + [pltpu.SemaphoreType.DMA] * 12,
      compiler_params=pltpu.CompilerParams(use_tc_tiling_on_sc=False,
                                           needs_layout_passes=False),
  )
  def sc_cell(idx_hbm, hall_hbm, auxt_hbm, gam_hbm, bet_hbm, out_hbm,
              idx_v, rows, w1t, w2t, own_v, outb, gam_v, bet_v, *sems):
    sgs = sems[0:4]
    sos = sems[4:8]
    sus = sems[8:12]
    wid = lax.axis_index("s") * 2 + lax.axis_index("c")
    base = wid * NPW
    pltpu.sync_copy(idx_hbm.at[wid], idx_v)
    pltpu.sync_copy(auxt_hbm.at[0], w1t)
    pltpu.sync_copy(auxt_hbm.at[1], w2t)
    pltpu.sync_copy(gam_hbm, gam_v)
    pltpu.sync_copy(bet_hbm, bet_v)

    def own_at(c):
      return hall_hbm.at[pl.ds(base + c * NB, NB)]

    def out_at(c):
      return out_hbm.at[pl.ds(base + c * NB, NB)]

    def process(c, q):
      pltpu.make_async_copy(own_at(c), own_v.at[q], sos[q]).wait()
      for b in range(NB):
        gidx = base + c * NB + b
        iv0 = idx_v[c, pl.ds(b * K, 16)]
        iv1 = idx_v[c, pl.ds(b * K + 16, 16)]
        w2s = plsc.load_gather(w2t, [jnp.full((16,), gidx, jnp.int32)])
        e0 = plsc.load_gather(w1t, [iv0]) + w2s
        e1 = plsc.load_gather(w1t, [iv1]) + w2s
        e0 = jnp.where(e0 > 0, e0, ALPHA * e0)
        e1 = jnp.where(e1 > 0, e1, ALPHA * e1)
        x0 = jnp.exp(e0)
        x1 = jnp.exp(e1)
        tot = _bsum(x0 + x1)
        a0 = x0 / tot
        a1 = x1 / tot
        acc = [jnp.zeros((16,), _f32) for _ in range(F2 // 16)]
        for k in range(K):
          wk = _lane_bcast(a0 if k < 16 else a1, k % 16)
          r = b * K + k
          for j in range(F2 // 16):
            acc[j] = acc[j] + wk * rows[q, r, pl.ds(j * 16, 16)]
        sv = jnp.zeros((16,), _f32)
        qv = jnp.zeros((16,), _f32)
        for j in range(F2 // 16):
          o = acc[j] + own_v[q, b, pl.ds(j * 16, 16)]
          o = jnp.where(o > 0, o, ALPHA * o)
          acc[j] = o
          sv = sv + o
          qv = qv + o * o
        mu = _bsum(sv) * (1.0 / F2)
        var = _bsum(qv) * (1.0 / F2) - mu * mu
        t = var + 1e-5
        ti = plsc.bitcast(t, jnp.int32)
        yi = jnp.int32(0x5F3759DF) - lax.shift_right_logical(ti, 1)
        y = plsc.bitcast(yi, _f32)
        for _ in range(3):
          y = y * (1.5 - 0.5 * t * y * y)
        for j in range(F2 // 16):
          g = gam_v[pl.ds(j * 16, 16)]
          bb = bet_v[pl.ds(j * 16, 16)]
          outb[q, b, pl.ds(j * 16, 16)] = (acc[j] - mu) * y * g + bb
      pltpu.async_copy(outb.at[q], out_at(c), sus[q])

    def gat(c, q):
      pltpu.async_copy(hall_hbm.at[idx_v.at[c]], rows.at[q], sgs[q])
      pltpu.async_copy(own_at(c), own_v.at[q], sos[q])

    gat(0, 0)
    gat(1, 1)
    gat(2, 2)

    def body(t, carry):
      for q in range(4):
        c = 4 * t + q

        @pl.when(c >= 4)
        def _(c=c, q=q):
          pltpu.make_async_copy(outb.at[q], out_at(c - 4), sus[q]).wait()

        pltpu.make_async_copy(hall_hbm.at[idx_v.at[c]], rows.at[q],
                              sgs[q]).wait()
        process(c, q)

        @pl.when(c + 3 < NCHUNK)
        def _(c=c, q=q):
          gat(c + 3, (q + 3) % 4)
      return carry

    lax.fori_loop(0, NCHUNK // 4, body, 0)
    for q in range(4):
      pltpu.make_async_copy(outb.at[q], out_at(NCHUNK - 4 + q), sus[q]).wait()

  return sc_cell


def _sc_cell(kadj_r, h_all, auxt, gam, bet):
  return _make_sc_cell()(kadj_r, h_all, auxt, gam, bet)


def _tcb_body(x_ref, hk_ref, r2_ref, wcol_ref, bcol_ref, wall_ref, ball_ref,
              c18_ref, hall_ref, auxt_ref, aux_scr):
  r2 = r2_ref[...]
  wcol = wcol_ref[...]
  bcol = bcol_ref[...]
  wall = wall_ref[...]
  ball = ball_ref[...]
  rown = lax.broadcasted_iota(jnp.int32, (S, S * F), 0)
  coln = lax.broadcasted_iota(jnp.int32, (S, S * F), 1) // F
  maskx = rown == coln
  onesbd = jnp.where(maskx, 1.0, 0.0).astype(_f32)

  def sub(i, carry):
    xs = x_ref[pl.ds(i * S, S), :]
    x_rep = jnp.dot(r2, xs, precision=_HI,
                    preferred_element_type=_f32)
    wht = jax.nn.relu(wcol * x_rep + bcol)
    s4dk = S4 * DK
    bigs = [
        jnp.dot(wall, wht[g * s4dk:(g + 1) * s4dk], precision=_HI,
                preferred_element_type=_f32) + ball
        for g in range(S // S4)
    ]
    q3 = jnp.concatenate([b[0:s4dk] for b in bigs],
                         axis=0).reshape(S, DK, F)
    k3 = jnp.concatenate([b[s4dk:2 * s4dk] for b in bigs],
                         axis=0).reshape(S, DK, F)
    p3 = jnp.concatenate([b[2 * s4dk:] for b in bigs],
                         axis=0).reshape(S, K, F)
    hk3 = hk_ref[pl.ds(i * S * K, S * K), :].reshape(S, K, F)

    lre = lax.dot_general(k3, q3, (((1,), (1,)), ((0,), (0,))),
                          precision=_HI, preferred_element_type=_f32)
    lcc = lax.dot_general(hk3, p3, (((1,), (1,)), ((0,), (0,))),
                          precision=_HI, preferred_element_type=_f32)
    ere = jnp.exp(lre.reshape(S * F, F))
    ecc = jnp.exp(lcc.reshape(S * F, F))
    xbd = jnp.where(maskx, jnp.tile(xs, (1, S)), 0.0)
    wsel = jnp.concatenate([xbd, onesbd], axis=0)
    outre = jnp.dot(wsel, ere, precision=_HI,
                    preferred_element_type=_f32)
    outcc = jnp.dot(wsel, ecc, precision=_HI,
                    preferred_element_type=_f32)
    hre = outre[0:S] / outre[S:2 * S] + xs
    hcc = outcc[0:S] / outcc[S:2 * S] + xs
    hall_s = jnp.concatenate([hre, hcc], axis=1)
    hall_ref[pl.ds(i * S, S), :] = hall_s
    aux = jnp.dot(hall_s, c18_ref[...], precision=_HI,
                  preferred_element_type=_f32)
    aux_scr[pl.ds(i * S, S), :] = aux
    return carry

  lax.fori_loop(0, G // S, sub, 0)
  auxt_ref[...] = lax.transpose(aux_scr[...], (1, 0))


def _tc_dense(xp, hk, r2, wcol, bcol, wall, ball, c18):
  nn = xp.shape[0]
  wspec = lambda shape: pl.BlockSpec(shape, lambda i: (0, 0))
  return pl.pallas_call(
      _tcb_body,
      grid=(nn // G,),
      in_specs=[
          pl.BlockSpec((G, F), lambda i: (i, 0)),
          pl.BlockSpec((G * K, F), lambda i: (i, 0)),
          wspec((S * DK, S)), wspec((S * DK, 1)), wspec((S * DK, 1)),
          wspec((S4 * (2 * DK + K), S4 * DK)), wspec((S4 * (2 * DK + K), 1)),
          wspec((F2, 8)),
      ],
      out_specs=[
          pl.BlockSpec((G, F2), lambda i: (i, 0)),
          pl.BlockSpec((8, G), lambda i: (0, i)),
      ],
      out_shape=[
          jax.ShapeDtypeStruct((nn, F2), _f32),
          jax.ShapeDtypeStruct((8, nn), _f32),
      ],
      scratch_shapes=[pltpu.VMEM((G, 8), _f32)],
  )(xp, hk, r2, wcol, bcol, wall, ball, c18)


def kernel(x, kadj, Wh_w, Wh_b, Wq, bq, Wk, bk, a_gene_cc, W_cell_cc,
           a_cell_cc, ln_gamma, ln_beta):
  x = x.astype(_f32)
  kadj = kadj.astype(jnp.int32)

  xp = jnp.zeros((NP, F), _f32).at[:N].set(x)
  kadjp = jnp.zeros((NP, K), jnp.int32).at[:N].set(kadj)
  kadj_r = kadjp.reshape(NW, NCHUNK, CHUNK)

  eye_s = jnp.eye(S, dtype=_f32)
  r2 = jnp.kron(eye_s, jnp.ones((DK, 1), _f32))
  wcol = jnp.tile(Wh_w[0], S)[:, None].astype(_f32)
  bcol = jnp.tile(Wh_b, S)[:, None].astype(_f32)
  wqtk = jnp.kron(eye_s, Wq.T.astype(_f32)) * INV_SCALE
  bqcol = (jnp.tile(bq, S)[:, None] * INV_SCALE).astype(_f32)
  wktk = jnp.kron(eye_s, Wk.T.astype(_f32))
  bkcol = jnp.tile(bk, S)[:, None].astype(_f32)
  agtk = jnp.kron(eye_s, a_gene_cc.T.astype(_f32))
  eye_s4 = jnp.eye(S4, dtype=_f32)
  wq4 = jnp.kron(eye_s4, Wq.T.astype(_f32)) * INV_SCALE
  wk4 = jnp.kron(eye_s4, Wk.T.astype(_f32))
  ag4 = jnp.kron(eye_s4, a_gene_cc.T.astype(_f32))
  bq4 = (jnp.tile(bq, S4)[:, None] * INV_SCALE).astype(_f32)
  bk4 = jnp.tile(bk, S4)[:, None].astype(_f32)
  wall = jnp.concatenate([wq4, wk4, ag4 @ wq4], axis=0)
  ball = jnp.concatenate([bq4, bk4, ag4 @ bq4], axis=0)

  c1 = (W_cell_cc @ a_cell_cc[:EMB_SPLIT]).astype(_f32)
  c2 = (W_cell_cc @ a_cell_cc[EMB_SPLIT:]).astype(_f32)
  c18 = jnp.concatenate([c1, c2, jnp.zeros((F2, 6), _f32)], axis=1)

  half = NP // 2
  nchunk_h = half * K // (NW * CHUNK)
  kadj_lo = kadjp[:half].reshape(NW, nchunk_h, CHUNK)
  kadj_hi = kadjp[half:].reshape(NW, nchunk_h, CHUNK)
  hk_lo = _sc_gather_x(kadj_lo, x)
  hk_hi = _sc_gather_x(kadj_hi, x)
  h_lo, aux_lo = _tc_dense(xp[:half], hk_lo, r2, wcol, bcol, wall, ball, c18)
  h_hi, aux_hi = _tc_dense(xp[half:], hk_hi, r2, wcol, bcol, wall, ball, c18)
  h_all = jnp.concatenate([h_lo, h_hi], axis=0)
  auxt = jnp.concatenate([aux_lo, aux_hi], axis=1)
  out = _sc_cell(kadj_r, h_all, auxt, ln_gamma.astype(_f32),
                 ln_beta.astype(_f32))
  return out[:N]


# speedup vs baseline: 1.0737x; 1.0009x over previous
import functools
import math

import jax
import jax.numpy as jnp
from jax import lax
from jax.experimental import pallas as pl
from jax.experimental.pallas import tpu as pltpu
from jax.experimental.pallas import tpu_sc as plsc

N = 10000
F = 64
K = 32
DK = 16
F2 = 2 * F
EMB_SPLIT = 64
ALPHA = 0.1
INV_SCALE = 1.0 / math.sqrt(DK)

NW = 32
NPW = 320
NP = NW * NPW
CHUNK = 128
NCHUNK = NPW * K // CHUNK

G = 256
S = 16
S4 = 4

_HI = jax.lax.Precision.DEFAULT
_f32 = jnp.float32


@functools.lru_cache(maxsize=None)
def _make_sc_gather(D, nchunk):
  mesh = plsc.VectorSubcoreMesh(core_axis_name="c", subcore_axis_name="s")

  @functools.partial(
      pl.kernel,
      out_type=jax.ShapeDtypeStruct((NW * nchunk * CHUNK, D), _f32),
      mesh=mesh,
      scratch_types=[
          pltpu.VMEM((nchunk, CHUNK), jnp.int32),
          pltpu.VMEM((4, CHUNK, D), _f32),
          pltpu.SemaphoreType.DMA,
          pltpu.SemaphoreType.DMA,
          pltpu.SemaphoreType.DMA,
          pltpu.SemaphoreType.DMA,
          pltpu.SemaphoreType.DMA,
          pltpu.SemaphoreType.DMA,
          pltpu.SemaphoreType.DMA,
          pltpu.SemaphoreType.DMA,
      ],
      compiler_params=pltpu.CompilerParams(use_tc_tiling_on_sc=False),
  )
  def sc_gather(idx_hbm, tab_hbm, out_hbm, idx_v, rows, sg0, sg1, sg2, sg3,
                ss0, ss1, ss2, ss3):
    wid = lax.axis_index("s") * 2 + lax.axis_index("c")
    pltpu.sync_copy(idx_hbm.at[wid], idx_v)
    base = wid * (nchunk * CHUNK)
    sgs = [sg0, sg1, sg2, sg3]
    sss = [ss0, ss1, ss2, ss3]

    def out_at(c):
      return out_hbm.at[pl.ds(base + c * CHUNK, CHUNK)]

    def body(t, carry):
      @pl.when(t > 0)
      def _():
        for q in range(4):
          pltpu.make_async_copy(rows.at[q], out_at(4 * t + q - 4),
                                sss[q]).wait()

      for q in range(4):
        pltpu.async_copy(tab_hbm.at[idx_v.at[4 * t + q]], rows.at[q], sgs[q])
      for q in range(4):
        c = 4 * t + q
        pltpu.make_async_copy(tab_hbm.at[idx_v.at[c]], rows.at[q],
                              sgs[q]).wait()
        pltpu.async_copy(rows.at[q], out_at(c), sss[q])
      return carry

    lax.fori_loop(0, nchunk // 4, body, 0)
    for q in range(4):
      pltpu.make_async_copy(rows.at[q], out_at(nchunk - 4 + q), sss[q]).wait()

  return sc_gather


def _sc_gather_x(kadj_r, tab):
  return _make_sc_gather(F, kadj_r.shape[1])(kadj_r, tab)


NB = 4


def _lane_bcast(v, lane):
  return lax.gather(
      v, jnp.full((16, 1), lane, jnp.int32),
      lax.GatherDimensionNumbers(offset_dims=(), collapsed_slice_dims=(0,),
                                 start_index_map=(0,)),
      (1,), mode=lax.GatherScatterMode.PROMISE_IN_BOUNDS)


def _bsum(v):
  return _lane_bcast(plsc.cumsum(v), 15)


@functools.lru_cache(maxsize=None)
def _make_sc_cell():
  mesh = plsc.VectorSubcoreMesh(core_axis_name="c", subcore_axis_name="s")

  @functools.partial(
      pl.kernel,
      out_type=jax.ShapeDtypeStruct((NP, F2), _f32),
      mesh=mesh,
      scratch_types=[
          pltpu.VMEM((NCHUNK, CHUNK), jnp.int32),
          pltpu.VMEM((4, CHUNK, F2), _f32),
          pltpu.VMEM((NP,), _f32),
          pltpu.VMEM((NP,), _f32),
          pltpu.VMEM((4, NB, F2), _f32),
          pltpu.VMEM((4, NB, F2), _f32),
          pltpu.VMEM((F2,), _f32),
          pltpu.VMEM((F2,), _f32),
      ] + [pltpu.SemaphoreType.DMA] * 12,
      compiler_params=pltpu.CompilerParams(use_tc_tiling_on_sc=False,
                                           needs_layout_passes=False),
  )
  def sc_cell(idx_hbm, hall_hbm, auxt_hbm, gam_hbm, bet_hbm, out_hbm,
              idx_v, rows, w1t, w2t, own_v, outb, gam_v, bet_v, *sems):
    sgs = sems[0:4]
    sos = sems[4:8]
    sus = sems[8:12]
    wid = lax.axis_index("s") * 2 + lax.axis_index("c")
    base = wid * NPW
    pltpu.sync_copy(idx_hbm.at[wid], idx_v)
    pltpu.sync_copy(auxt_hbm.at[0], w1t)
    pltpu.sync_copy(auxt_hbm.at[1], w2t)
    pltpu.sync_copy(gam_hbm, gam_v)
    pltpu.sync_copy(bet_hbm, bet_v)

    def own_at(c):
      return hall_hbm.at[pl.ds(base + c * NB, NB)]

    def out_at(c):
      return out_hbm.at[pl.ds(base + c * NB, NB)]

    def process(c, q):
      pltpu.make_async_copy(own_at(c), own_v.at[q], sos[q]).wait()
      for b in range(NB):
        gidx = base + c * NB + b
        iv0 = idx_v[c, pl.ds(b * K, 16)]
        iv1 = idx_v[c, pl.ds(b * K + 16, 16)]
        w2s = plsc.load_gather(w2t, [jnp.full((16,), gidx, jnp.int32)])
        e0 = plsc.load_gather(w1t, [iv0]) + w2s
        e1 = plsc.load_gather(w1t, [iv1]) + w2s
        e0 = jnp.where(e0 > 0, e0, ALPHA * e0)
        e1 = jnp.where(e1 > 0, e1, ALPHA * e1)
        x0 = jnp.exp(e0)
        x1 = jnp.exp(e1)
        tot = _bsum(x0 + x1)
        a0 = x0 / tot
        a1 = x1 / tot
        acc = [jnp.zeros((16,), _f32) for _ in range(F2 // 16)]
        for k in range(K):
          wk = _lane_bcast(a0 if k < 16 else a1, k % 16)
          r = b * K + k
          for j in range(F2 // 16):
            acc[j] = acc[j] + wk * rows[q, r, pl.ds(j * 16, 16)]
        sv = jnp.zeros((16,), _f32)
        qv = jnp.zeros((16,), _f32)
        for j in range(F2 // 16):
          o = acc[j] + own_v[q, b, pl.ds(j * 16, 16)]
          o = jnp.where(o > 0, o, ALPHA * o)
          acc[j] = o
          sv = sv + o
          qv = qv + o * o
        mu = _bsum(sv) * (1.0 / F2)
        var = _bsum(qv) * (1.0 / F2) - mu * mu
        t = var + 1e-5
        ti = plsc.bitcast(t, jnp.int32)
        yi = jnp.int32(0x5F3759DF) - lax.shift_right_logical(ti, 1)
        y = plsc.bitcast(yi, _f32)
        for _ in range(3):
          y = y * (1.5 - 0.5 * t * y * y)
        for j in range(F2 // 16):
          g = gam_v[pl.ds(j * 16, 16)]
          bb = bet_v[pl.ds(j * 16, 16)]
          outb[q, b, pl.ds(j * 16, 16)] = (acc[j] - mu) * y * g + bb
      pltpu.async_copy(outb.at[q], out_at(c), sus[q])

    def gat(c, q):
      pltpu.async_copy(hall_hbm.at[idx_v.at[c]], rows.at[q], sgs[q])
      pltpu.async_copy(own_at(c), own_v.at[q], sos[q])

    gat(0, 0)
    gat(1, 1)
    gat(2, 2)

    def body(t, carry):
      for q in range(4):
        c = 4 * t + q

        @pl.when(c >= 4)
        def _(c=c, q=q):
          pltpu.make_async_copy(outb.at[q], out_at(c - 4), sus[q]).wait()

        pltpu.make_async_copy(hall_hbm.at[idx_v.at[c]], rows.at[q],
                              sgs[q]).wait()
        process(c, q)

        @pl.when(c + 3 < NCHUNK)
        def _(c=c, q=q):
          gat(c + 3, (q + 3) % 4)
      return carry

    lax.fori_loop(0, NCHUNK // 4, body, 0)
    for q in range(4):
      pltpu.make_async_copy(outb.at[q], out_at(NCHUNK - 4 + q), sus[q]).wait()

  return sc_cell


def _sc_cell(kadj_r, h_all, auxt, gam, bet):
  return _make_sc_cell()(kadj_r, h_all, auxt, gam, bet)


def _tcb_body(x_ref, hk_ref, r2_ref, wcol_ref, bcol_ref, wall_ref, ball_ref,
              c18_ref, hall_ref, auxt_ref, aux_scr):
  r2 = r2_ref[...]
  wcol = wcol_ref[...]
  bcol = bcol_ref[...]
  wall = wall_ref[...]
  ball = ball_ref[...]
  rown = lax.broadcasted_iota(jnp.int32, (S, S * F), 0)
  coln = lax.broadcasted_iota(jnp.int32, (S, S * F), 1) // F
  maskx = rown == coln
  onesbd = jnp.where(maskx, 1.0, 0.0).astype(_f32)

  def sub(i, carry):
    xs = x_ref[pl.ds(i * S, S), :]
    x_rep = jnp.dot(r2, xs, precision=_HI,
                    preferred_element_type=_f32)
    wht = jax.nn.relu(wcol * x_rep + bcol)
    s4dk = S4 * DK
    bigs = [
        jnp.dot(wall, wht[g * s4dk:(g + 1) * s4dk], precision=_HI,
                preferred_element_type=_f32) + ball
        for g in range(S // S4)
    ]
    q3 = jnp.concatenate([b[0:s4dk] for b in bigs],
                         axis=0).reshape(S, DK, F)
    k3 = jnp.concatenate([b[s4dk:2 * s4dk] for b in bigs],
                         axis=0).reshape(S, DK, F)
    p3 = jnp.concatenate([b[2 * s4dk:] for b in bigs],
                         axis=0).reshape(S, K, F)
    hk3 = hk_ref[pl.ds(i * S * K, S * K), :].reshape(S, K, F)

    lre = lax.dot_general(k3, q3, (((1,), (1,)), ((0,), (0,))),
                          precision=_HI, preferred_element_type=_f32)
    lcc = lax.dot_general(hk3, p3, (((1,), (1,)), ((0,), (0,))),
                          precision=_HI, preferred_element_type=_f32)
    ere = jnp.exp(lre.reshape(S * F, F))
    ecc = jnp.exp(lcc.reshape(S * F, F))
    xbd = jnp.where(maskx, jnp.tile(xs, (1, S)), 0.0)
    wsel = jnp.concatenate([xbd, onesbd], axis=0)
    outre = jnp.dot(wsel, ere, precision=_HI,
                    preferred_element_type=_f32)
    outcc = jnp.dot(wsel, ecc, precision=_HI,
                    preferred_element_type=_f32)
    hre = outre[0:S] / outre[S:2 * S] + xs
    hcc = outcc[0:S] / outcc[S:2 * S] + xs
    hall_s = jnp.concatenate([hre, hcc], axis=1)
    hall_ref[pl.ds(i * S, S), :] = hall_s
    aux = jnp.dot(hall_s, c18_ref[...], precision=_HI,
                  preferred_element_type=_f32)
    aux_scr[pl.ds(i * S, S), :] = aux
    return carry

  lax.fori_loop(0, G // S, sub, 0)
  auxt_ref[...] = lax.transpose(aux_scr[...], (1, 0))


def _tc_dense(xp, hk, r2, wcol, bcol, wall, ball, c18):
  nn = xp.shape[0]
  wspec = lambda shape: pl.BlockSpec(shape, lambda i: (0, 0))
  return pl.pallas_call(
      _tcb_body,
      grid=(nn // G,),
      in_specs=[
          pl.BlockSpec((G, F), lambda i: (i, 0)),
          pl.BlockSpec((G * K, F), lambda i: (i, 0)),
          wspec((S * DK, S)), wspec((S * DK, 1)), wspec((S * DK, 1)),
          wspec((S4 * (2 * DK + K), S4 * DK)), wspec((S4 * (2 * DK + K), 1)),
          wspec((F2, 8)),
      ],
      out_specs=[
          pl.BlockSpec((G, F2), lambda i: (i, 0)),
          pl.BlockSpec((8, G), lambda i: (0, i)),
      ],
      out_shape=[
          jax.ShapeDtypeStruct((nn, F2), _f32),
          jax.ShapeDtypeStruct((8, nn), _f32),
      ],
      scratch_shapes=[pltpu.VMEM((G, 8), _f32)],
  )(xp, hk, r2, wcol, bcol, wall, ball, c18)


def kernel(x, kadj, Wh_w, Wh_b, Wq, bq, Wk, bk, a_gene_cc, W_cell_cc,
           a_cell_cc, ln_gamma, ln_beta):
  x = x.astype(_f32)
  kadj = kadj.astype(jnp.int32)

  xp = jnp.zeros((NP, F), _f32).at[:N].set(x)
  kadjp = jnp.zeros((NP, K), jnp.int32).at[:N].set(kadj)
  kadj_r = kadjp.reshape(NW, NCHUNK, CHUNK)

  r2 = jnp.kron(jnp.eye(S, dtype=_f32), jnp.ones((DK, 1), _f32))
  wcol = jnp.tile(Wh_w[0], S)[:, None].astype(_f32)
  bcol = jnp.tile(Wh_b, S)[:, None].astype(_f32)
  eye_s4 = jnp.eye(S4, dtype=_f32)
  wq4 = jnp.kron(eye_s4, Wq.T.astype(_f32)) * INV_SCALE
  wk4 = jnp.kron(eye_s4, Wk.T.astype(_f32))
  ag4 = jnp.kron(eye_s4, a_gene_cc.T.astype(_f32))
  bq4 = (jnp.tile(bq, S4)[:, None] * INV_SCALE).astype(_f32)
  bk4 = jnp.tile(bk, S4)[:, None].astype(_f32)
  wall = jnp.concatenate([wq4, wk4, ag4 @ wq4], axis=0)
  ball = jnp.concatenate([bq4, bk4, ag4 @ bq4], axis=0)

  c1 = (W_cell_cc @ a_cell_cc[:EMB_SPLIT]).astype(_f32)
  c2 = (W_cell_cc @ a_cell_cc[EMB_SPLIT:]).astype(_f32)
  c18 = jnp.concatenate([c1, c2, jnp.zeros((F2, 6), _f32)], axis=1)

  half = NP // 2
  nchunk_h = half * K // (NW * CHUNK)
  kadj_lo = kadjp[:half].reshape(NW, nchunk_h, CHUNK)
  kadj_hi = kadjp[half:].reshape(NW, nchunk_h, CHUNK)
  hk_lo = _sc_gather_x(kadj_lo, x)
  hk_hi = _sc_gather_x(kadj_hi, x)
  h_lo, aux_lo = _tc_dense(xp[:half], hk_lo, r2, wcol, bcol, wall, ball, c18)
  h_hi, aux_hi = _tc_dense(xp[half:], hk_hi, r2, wcol, bcol, wall, ball, c18)
  h_all = jnp.concatenate([h_lo, h_hi], axis=0)
  auxt = jnp.concatenate([aux_lo, aux_hi], axis=1)
  out = _sc_cell(kadj_r, h_all, auxt, ln_gamma.astype(_f32),
                 ln_beta.astype(_f32))
  return out[:N]
